# baseline (speedup 1.0000x reference)
import functools

import jax
import jax.numpy as jnp
from jax import lax
from jax.experimental import pallas as pl
from jax.experimental.pallas import tpu as pltpu
from jax.experimental.pallas import tpu_sc as plsc

VOCAB = 1000000
N_EMB = 64
BATCH = 16384
HIST = 50

_B_FLAT = BATCH * HIST
_CHUNK = 128
_NW = 32
_TASKS = _B_FLAT // _CHUNK
_TASKS_PER_W = _TASKS // _NW

_NBUF = 4
_DEPTH = 3
_GROUPS = _TASKS_PER_W // _NBUF


def _make_gather():
    mesh = plsc.VectorSubcoreMesh(core_axis_name="c", subcore_axis_name="s")

    @functools.partial(
        pl.kernel,
        mesh=mesh,
        out_type=jax.ShapeDtypeStruct((HIST, 8, BATCH // _CHUNK, 8, _CHUNK),
                                      jnp.float32),
        scratch_types=[
            pltpu.VMEM((_TASKS_PER_W, _CHUNK), jnp.int32),
            pltpu.VMEM((_NBUF, _CHUNK, N_EMB), jnp.float32),
            pltpu.VMEM((_NBUF, 8, 8, _CHUNK), jnp.float32),
            pltpu.SemaphoreType.DMA,
            pltpu.SemaphoreType.DMA,
        ],
        compiler_params=pltpu.CompilerParams(use_tc_tiling_on_sc=False,
                                             needs_layout_passes=False),
    )
    def gather_kernel(table_hbm, idx_hbm, out_hbm, idx_v, rows_v, tiles_v,
                      gsem, osem):
        wid = lax.axis_index("s") * 2 + lax.axis_index("c")
        task0 = wid * _TASKS_PER_W

        pltpu.sync_copy(idx_hbm.at[pl.ds(task0, _TASKS_PER_W)], idx_v)

        for b in range(_DEPTH):
            pltpu.async_copy(table_hbm.at[idx_v.at[b]], rows_v.at[b], gsem)

        def out_slice(task):
            h = task // (BATCH // _CHUNK)
            bb = lax.rem(task, BATCH // _CHUNK)
            return out_hbm.at[h, :, bb, :, :]

        lane = lax.iota(jnp.int32, 16)
        bases = [lane + (bc * 16) for bc in range(8)]

        def transpose_task(rref, tref):
            @plsc.parallel_loop(0, N_EMB, unroll=4)
            def _(c):
                cvec = lax.broadcast(c, (16,))
                vals = [plsc.load_gather(rref, [bases[bc], cvec])
                        for bc in range(8)]
                cb = lax.shift_right_logical(c, 3)
                c8 = lax.bitwise_and(c, 7)
                for bc in range(8):
                    tref[cb, c8, pl.ds(bc * 16, 16)] = vals[bc]

        def group(g, _):
            for b in range(_NBUF):
                j = g * _NBUF + b
                task = task0 + j
                gbuf = (b + _DEPTH) % _NBUF

                @pl.when(j + _DEPTH - _NBUF >= 0)
                def _():
                    pltpu.make_async_copy(
                        tiles_v.at[gbuf],
                        out_slice(task + _DEPTH - _NBUF), osem).wait()

                @pl.when(j + _DEPTH < _TASKS_PER_W)
                def _():
                    pltpu.async_copy(table_hbm.at[idx_v.at[j + _DEPTH]],
                                     rows_v.at[gbuf], gsem)

                pltpu.make_async_copy(table_hbm.at[idx_v.at[j]],
                                      rows_v.at[b], gsem).wait()
                transpose_task(rows_v.at[b], tiles_v.at[b])
                pltpu.async_copy(tiles_v.at[b], out_slice(task), osem)
            return ()

        lax.fori_loop(0, _GROUPS, group, (), unroll=False)

        for t in range(_TASKS_PER_W + _DEPTH - _NBUF, _TASKS_PER_W):
            pltpu.make_async_copy(tiles_v.at[t % _NBUF],
                                  out_slice(task0 + t), osem).wait()

    return gather_kernel


_gather = _make_gather()


def kernel(idx, table):
    idx_t = idx.astype(jnp.int32).T.reshape(_TASKS, _CHUNK)
    out5d = _gather(table, idx_t)
    t1 = jnp.transpose(out5d, (2, 4, 1, 3, 0))
    r1 = t1.reshape(BATCH, N_EMB, HIST)
    return jnp.transpose(r1, (0, 2, 1))


# speedup vs baseline: 1.8332x; 1.8332x over previous
import functools

import jax
import jax.numpy as jnp
from jax import lax
from jax.experimental import pallas as pl
from jax.experimental.pallas import tpu as pltpu
from jax.experimental.pallas import tpu_sc as plsc

VOCAB = 1000000
N_EMB = 64
BATCH = 16384
HIST = 50

_B_FLAT = BATCH * HIST
_CHUNK = 128
_NW = 32
_TASKS = _B_FLAT // _CHUNK
_TASKS_PER_W = _TASKS // _NW

_NBUF = 4
_DEPTH = 3
_GROUPS = _TASKS_PER_W // _NBUF


def _make_gather():
    mesh = plsc.VectorSubcoreMesh(core_axis_name="c", subcore_axis_name="s")

    @functools.partial(
        pl.kernel,
        mesh=mesh,
        out_type=jax.ShapeDtypeStruct((HIST, 8, BATCH // _CHUNK, 8, _CHUNK),
                                      jnp.float32),
        scratch_types=[
            pltpu.VMEM((_TASKS_PER_W, _CHUNK), jnp.int32),
            pltpu.VMEM((_NBUF, _CHUNK, N_EMB), jnp.float32),
            pltpu.VMEM((_NBUF, 8, 8, _CHUNK + 1), jnp.float32),
            pltpu.SemaphoreType.DMA,
            pltpu.SemaphoreType.DMA,
        ],
        compiler_params=pltpu.CompilerParams(use_tc_tiling_on_sc=False,
                                             needs_layout_passes=False),
    )
    def gather_kernel(table_hbm, idx_hbm, out_hbm, idx_v, rows_v, tiles_v,
                      gsem, osem):
        wid = lax.axis_index("s") * 2 + lax.axis_index("c")
        task0 = wid * _TASKS_PER_W

        pltpu.sync_copy(idx_hbm.at[pl.ds(task0, _TASKS_PER_W)], idx_v)

        for b in range(_DEPTH):
            pltpu.async_copy(table_hbm.at[idx_v.at[b]], rows_v.at[b], gsem)

        def out_slice(task):
            h = task // (BATCH // _CHUNK)
            bb = lax.rem(task, BATCH // _CHUNK)
            return out_hbm.at[h, :, bb, :, :]

        lane = lax.iota(jnp.int32, 16)
        i0 = [lax.shift_right_logical(lane + cc * 16, 3) for cc in range(4)]
        i1 = [lax.bitwise_and(lane + cc * 16, 7) for cc in range(4)]

        def transpose_task(rref, tref):
            @plsc.parallel_loop(0, _CHUNK, unroll=4)
            def _(bl):
                blv = lax.broadcast(bl, (16,))
                vals = [rref[bl, pl.ds(cc * 16, 16)] for cc in range(4)]
                for cc in range(4):
                    plsc.store_scatter(tref, [i0[cc], i1[cc], blv], vals[cc])

        def group(g, _):
            for b in range(_NBUF):
                j = g * _NBUF + b
                task = task0 + j
                gbuf = (b + _DEPTH) % _NBUF

                @pl.when(j + _DEPTH - _NBUF >= 0)
                def _():
                    pltpu.make_async_copy(
                        tiles_v.at[gbuf, :, :, pl.ds(0, _CHUNK)],
                        out_slice(task + _DEPTH - _NBUF), osem).wait()

                @pl.when(j + _DEPTH < _TASKS_PER_W)
                def _():
                    pltpu.async_copy(table_hbm.at[idx_v.at[j + _DEPTH]],
                                     rows_v.at[gbuf], gsem)

                pltpu.make_async_copy(table_hbm.at[idx_v.at[j]],
                                      rows_v.at[b], gsem).wait()
                transpose_task(rows_v.at[b], tiles_v.at[b])
                pltpu.async_copy(tiles_v.at[b, :, :, pl.ds(0, _CHUNK)],
                                 out_slice(task), osem)
            return ()

        lax.fori_loop(0, _GROUPS, group, (), unroll=False)

        for t in range(_TASKS_PER_W + _DEPTH - _NBUF, _TASKS_PER_W):
            pltpu.make_async_copy(tiles_v.at[t % _NBUF, :, :, pl.ds(0, _CHUNK)],
                                  out_slice(task0 + t), osem).wait()

    return gather_kernel


_gather = _make_gather()


def kernel(idx, table):
    idx_t = idx.astype(jnp.int32).T.reshape(_TASKS, _CHUNK)
    out5d = _gather(table, idx_t)
    t1 = jnp.transpose(out5d, (2, 4, 1, 3, 0))
    r1 = t1.reshape(BATCH, N_EMB, HIST)
    return jnp.transpose(r1, (0, 2, 1))
